# baseline (speedup 1.0000x reference)
import dataclasses
import functools

import jax
import jax.numpy as jnp
from jax import lax
from jax.experimental import pallas as pl
from jax.experimental.pallas import tpu as pltpu
from jax.experimental.pallas import tpu_sc as plsc

NV = 10000
NE = 10000
NNZ = 320000
D = 128
DH = 64
NS = 16
CHUNK = 80
PAIRS_PER_TILE = NNZ // NS
CHUNKS_PER_TILE = PAIRS_PER_TILE // CHUNK
NPAD = 10240
STRIPE = NPAD // NS
NROWS = NPAD // 16
RB = 1000

_MESH = plsc.VectorSubcoreMesh(core_axis_name="c", subcore_axis_name="s")

_SC_PARAMS = pltpu.CompilerParams()
for _f, _v in (("needs_layout_passes", False), ("use_tc_tiling_on_sc", False)):
    if _f in pltpu.CompilerParams.__dataclass_fields__:
        _SC_PARAMS = dataclasses.replace(_SC_PARAMS, **{_f: _v})


@functools.partial(
    pl.kernel,
    out_type=jax.ShapeDtypeStruct((2, NROWS, 16), jnp.float32),
    mesh=_MESH,
    compiler_params=_SC_PARAMS,
    scratch_types=[
        pltpu.VMEM((CHUNKS_PER_TILE, CHUNK), jnp.int32),
        pltpu.VMEM((NROWS, 16), jnp.float32),
        pltpu.VMEM((NROWS, 16), jnp.float32),
        pltpu.VMEM((5, 128), jnp.int32),
        pltpu.VMEM_SHARED((NROWS, 16), jnp.float32),
        pltpu.VMEM_SHARED((NROWS, 16), jnp.float32),
    ],
)
def _deg_kernel(nidx3, eidx3, rout, idx_t, dega_t, degb_t, ramp_t,
                acca_s, accb_s):
    c = lax.axis_index("c")
    s = lax.axis_index("s")
    z16 = jnp.zeros((16,), jnp.float32)
    ones16 = jnp.ones((16,), jnp.float32)
    iota16 = lax.iota(jnp.int32, 16)

    @pl.loop(0, NROWS)
    def _(r):
        dega_t[r, :] = z16
        degb_t[r, :] = z16

    pltpu.sync_copy(dega_t.at[pl.ds(0, 40)], acca_s.at[pl.ds(s * 40, 40)])
    pltpu.sync_copy(degb_t.at[pl.ds(0, 40)], accb_s.at[pl.ds(s * 40, 40)])
    plsc.subcore_barrier()

    def _hist(idx3, deg_t):
        pltpu.sync_copy(idx3.at[s], idx_t)

        @pl.loop(0, CHUNKS_PER_TILE)
        def _(r):
            for k in range(CHUNK // 16):
                v = idx_t[r, pl.ds(k * 16, 16)]
                plsc.addupdate_scatter(
                    deg_t,
                    [lax.shift_right_logical(v, 4), lax.bitwise_and(v, 15)],
                    ones16,
                )

    _hist(eidx3, dega_t)
    _hist(nidx3, degb_t)

    for rr in range(5):
        for jj in range(8):
            ramp_t[rr, pl.ds(jj * 16, 16)] = iota16 + (rr * 128 + jj * 16)
    for rr in range(5):
        pltpu.sync_copy(dega_t.at[pl.ds(rr * 128, 128)],
                        acca_s.at[ramp_t.at[rr]], add=True)
        pltpu.sync_copy(degb_t.at[pl.ds(rr * 128, 128)],
                        accb_s.at[ramp_t.at[rr]], add=True)
    plsc.subcore_barrier()

    def _recip(acc_s, deg_t, kind):
        pltpu.sync_copy(acc_s.at[pl.ds(s * 40, 40)], deg_t.at[pl.ds(0, 40)])

        @pl.loop(0, 40)
        def _(r):
            deg_t[r, :] = 1.0 / jnp.maximum(deg_t[r, :], 1.0)

        pltpu.sync_copy(deg_t.at[pl.ds(0, 40)],
                        rout.at[kind].at[pl.ds(s * 40, 40)])

    @pl.when(c == 0)
    def _():
        _recip(acca_s, dega_t, 0)

    @pl.when(c == 1)
    def _():
        _recip(accb_s, degb_t, 1)


@functools.partial(
    pl.kernel,
    out_type=[jax.ShapeDtypeStruct((2, NPAD, DH), jnp.float32),
              jax.ShapeDtypeStruct((2, NPAD, DH), jnp.float32)],
    mesh=_MESH,
    compiler_params=_SC_PARAMS,
    scratch_types=[
        pltpu.VMEM((CHUNKS_PER_TILE, CHUNK), jnp.int32),
        pltpu.VMEM((CHUNKS_PER_TILE, CHUNK), jnp.int32),
        pltpu.VMEM((CHUNK, DH), jnp.float32),
        pltpu.VMEM((CHUNK, DH), jnp.float32),
        pltpu.VMEM((CHUNK, DH), jnp.float32),
        pltpu.VMEM((CHUNK, DH), jnp.float32),
        pltpu.VMEM((CHUNK, DH), jnp.float32),
        pltpu.VMEM((CHUNK, DH), jnp.float32),
        pltpu.VMEM((CHUNK, DH), jnp.float32),
        pltpu.VMEM((CHUNK, DH), jnp.float32),
        pltpu.VMEM((1, STRIPE), jnp.float32),
        pltpu.VMEM_SHARED((NPAD, DH), jnp.float32),
    ] + [pltpu.SemaphoreType.DMA] * 11,
)
def _conv_kernel(h2, nidx3, eidx3, recip_e, recip_v, topoh, out2, e_out,
                 nidx_t, eidx_t, r0, r1, r2, r3, r4, tmp_t, ttmp_t, zeros_t,
                 recip_t, acc, g0, g1, g2, g3, g4, s0, s1, s2, s3, s4, zs):
    c = lax.axis_index("c")
    s = lax.axis_index("s")
    z16 = jnp.zeros((16,), jnp.float32)
    z16i = jnp.zeros((16,), jnp.int32)
    NCH = STRIPE // CHUNK

    @pl.loop(0, CHUNK)
    def _(r):
        for k in range(DH // 16):
            zeros_t[r, pl.ds(k * 16, 16)] = z16

    def _zero_chunk_start(j):
        pltpu.async_copy(zeros_t, acc.at[pl.ds(s * STRIPE + j * CHUNK, CHUNK)],
                         zs)

    def _zero_drain(n):
        for _ in range(n):
            pltpu.make_async_copy(
                zeros_t, acc.at[pl.ds(s * STRIPE, CHUNK)], zs).wait()

    for j in range(NCH):
        _zero_chunk_start(j)
    pltpu.sync_copy(nidx3.at[s], nidx_t)
    pltpu.sync_copy(eidx3.at[s], eidx_t)
    _zero_drain(NCH)
    plsc.subcore_barrier()

    NB = 5
    BUFS = (r0, r1, r2, r3, r4)
    GSEMS = (g0, g1, g2, g3, g4)
    SSEMS = (s0, s1, s2, s3, s4)
    NITER = CHUNKS_PER_TILE // NB

    def _pipelined_pass(src_ref, gidx_t, dst_acc, sidx_t):
        def g_start(b, k):
            pltpu.async_copy(src_ref.at[gidx_t.at[k]], BUFS[b], GSEMS[b])

        def g_wait(b):
            pltpu.make_async_copy(
                src_ref.at[gidx_t.at[0]], BUFS[b], GSEMS[b]).wait()

        def s_start(b, k):
            pltpu.async_copy(BUFS[b], dst_acc.at[sidx_t.at[k]], SSEMS[b],
                             add=True)

        def s_wait(b):
            pltpu.make_async_copy(
                BUFS[b], dst_acc.at[sidx_t.at[0]], SSEMS[b]).wait()

        for b in range(NB):
            g_start(b, b)

        @pl.loop(0, NITER)
        def _(i):
            k = i * NB
            for b in range(NB):
                g_wait(b)
                s_start(b, k + b)

            @pl.when(i < NITER - 1)
            def _():
                for b in range(NB):
                    s_wait(b)
                    g_start(b, k + NB + b)

        for b in range(NB):
            s_wait(b)

    def _scale_rows(buf, j):
        @pl.loop(0, CHUNK)
        def _(r):
            rec = plsc.load_gather(
                recip_t, [z16i, jnp.full((16,), j * CHUNK + r, jnp.int32)])
            for k in range(DH // 16):
                buf[r, pl.ds(k * 16, 16)] = buf[r, pl.ds(k * 16, 16)] * rec

    def _combine_rows(buf, tbuf, j):
        @pl.loop(0, CHUNK)
        def _(r):
            rec = plsc.load_gather(
                recip_t, [z16i, jnp.full((16,), j * CHUNK + r, jnp.int32)])
            for k in range(DH // 16):
                v = buf[r, pl.ds(k * 16, 16)] * rec
                t = tbuf[r, pl.ds(k * 16, 16)]
                buf[r, pl.ds(k * 16, 16)] = jnp.maximum(v + v * t, 0.0)

    def _normalize_phase(dst_ref, fuse_zero, topo_src=None):
        bufs = (tmp_t, ttmp_t)
        tbufs = (r0, r1)
        rsems = (g0, g1)
        tsems = (g2, g3)
        wsems = (s0, s1)

        def chunk_slice(ref, j):
            return ref.at[pl.ds(s * STRIPE + j * CHUNK, CHUNK)]

        def r_start(j):
            pltpu.async_copy(chunk_slice(acc, j), bufs[j % 2], rsems[j % 2])
            if topo_src is not None:
                pltpu.async_copy(chunk_slice(topo_src, j), tbufs[j % 2],
                                 tsems[j % 2])

        def r_wait(j):
            pltpu.make_async_copy(
                chunk_slice(acc, j), bufs[j % 2], rsems[j % 2]).wait()
            if topo_src is not None:
                pltpu.make_async_copy(
                    chunk_slice(topo_src, j), tbufs[j % 2],
                    tsems[j % 2]).wait()

        def w_start(j):
            pltpu.async_copy(bufs[j % 2], chunk_slice(dst_ref, j),
                             wsems[j % 2])

        def w_wait(j):
            pltpu.make_async_copy(bufs[j % 2], chunk_slice(dst_ref, j),
                                  wsems[j % 2]).wait()

        r_start(0)
        for j in range(NCH):
            r_wait(j)
            if fuse_zero:
                _zero_chunk_start(j)
            if j + 1 < NCH:
                if j >= 1:
                    w_wait(j - 1)
                r_start(j + 1)
            if topo_src is None:
                _scale_rows(bufs[j % 2], j)
            else:
                _combine_rows(bufs[j % 2], tbufs[j % 2], j)
            w_start(j)
        w_wait(NCH - 2)
        w_wait(NCH - 1)
        if fuse_zero:
            _zero_drain(NCH)

    hc = h2.at[c]
    _pipelined_pass(hc, nidx_t, acc, eidx_t)
    plsc.subcore_barrier()

    ec = e_out.at[c]
    pltpu.sync_copy(recip_e.at[s], recip_t)
    _normalize_phase(ec, fuse_zero=True)
    plsc.subcore_barrier()

    _pipelined_pass(ec, eidx_t, acc, nidx_t)
    plsc.subcore_barrier()

    pltpu.sync_copy(recip_v.at[s], recip_t)
    _normalize_phase(out2.at[c], fuse_zero=False, topo_src=topoh.at[c])


def _mm(a, b):
    return jnp.dot(a, b, preferred_element_type=jnp.float32,
                   precision=lax.Precision.HIGHEST)


def _tpro_body(pd_ref, tW_ref, tb_ref, x_ref, Wg_ref, bg_ref,
               Wt0_ref, bt0_ref, Wt1_ref, bt1_ref, Wt2_ref, bt2_ref,
               t1h_ref, t2h_ref, t3h_ref, t3_ref, h2_ref):
    topo0 = jnp.maximum(_mm(pd_ref[...], tW_ref[...]) + tb_ref[...], 0.0)
    t1 = _mm(topo0, Wt0_ref[...]) + bt0_ref[...]
    t1h_ref[0] = t1[:, :DH]
    t1h_ref[1] = t1[:, DH:]
    t2 = _mm(t1, Wt1_ref[...]) + bt1_ref[...]
    t2h_ref[0] = t2[:, :DH]
    t2h_ref[1] = t2[:, DH:]
    t3 = _mm(t2, Wt2_ref[...]) + bt2_ref[...]
    t3_ref[...] = t3
    t3h_ref[0] = t3[:, :DH]
    t3h_ref[1] = t3[:, DH:]
    h = _mm(x_ref[...], Wg_ref[...]) + bg_ref[...]
    h2_ref[0] = h[:, :DH]
    h2_ref[1] = h[:, DH:]


def _tmm_body(xh_ref, Wg_ref, bg_ref, h2_ref):
    xfull = jnp.concatenate([xh_ref[0], xh_ref[1]], axis=1)
    h = _mm(xfull, Wg_ref[...]) + bg_ref[...]
    h2_ref[0] = h[:, :DH]
    h2_ref[1] = h[:, DH:]


_b_mat = pl.BlockSpec((D, D), lambda i: (0, 0))
_b_bias = pl.BlockSpec((1, D), lambda i: (0, 0))
_b_row = pl.BlockSpec((RB, D), lambda i: (i, 0))
_b_h2 = pl.BlockSpec((2, RB, DH), lambda i: (0, i, 0))

_PADH = jax.ShapeDtypeStruct((2, NPAD, DH), jnp.float32)

_tpro = pl.pallas_call(
    _tpro_body,
    grid=(NV // RB,),
    in_specs=[pl.BlockSpec((RB, 5), lambda i: (i, 0)),
              pl.BlockSpec((5, D), lambda i: (0, 0)),
              _b_bias, _b_row, _b_mat, _b_bias,
              _b_mat, _b_bias, _b_mat, _b_bias, _b_mat, _b_bias],
    out_specs=[_b_h2, _b_h2, _b_h2, _b_row, _b_h2],
    out_shape=[_PADH, _PADH, _PADH,
               jax.ShapeDtypeStruct((NV, D), jnp.float32), _PADH],
)

_tmm = pl.pallas_call(
    _tmm_body,
    grid=(NV // RB,),
    in_specs=[_b_h2, _b_mat, _b_bias],
    out_specs=_b_h2,
    out_shape=_PADH,
)


def kernel(x, hg, pd, tW, tb, Wg0, bg0, Wt0, bt0, Wg1, bg1, Wt1, bt1,
           Wg2, bg2, Wt2, bt2):
    nidx3 = hg[0].reshape(NS, CHUNKS_PER_TILE, CHUNK)
    eidx3 = hg[1].reshape(NS, CHUNKS_PER_TILE, CHUNK)
    rout = _deg_kernel(nidx3, eidx3)
    recip_e = rout[0].reshape(NS, 1, STRIPE)
    recip_v = rout[1].reshape(NS, 1, STRIPE)

    tb_ = tb.reshape(1, D)
    bg0_, bt0_ = bg0.reshape(1, D), bt0.reshape(1, D)
    bg1_, bt1_ = bg1.reshape(1, D), bt1.reshape(1, D)
    bg2_, bt2_ = bg2.reshape(1, D), bt2.reshape(1, D)

    t1h, t2h, t3h, topo3, h2 = _tpro(pd, tW, tb_, x, Wg0, bg0_,
                                     Wt0, bt0_, Wt1, bt1_, Wt2, bt2_)
    x1h, _ = _conv_kernel(h2, nidx3, eidx3, recip_e, recip_v, t1h)
    h2 = _tmm(x1h, Wg1, bg1_)
    x2h, _ = _conv_kernel(h2, nidx3, eidx3, recip_e, recip_v, t2h)
    h2 = _tmm(x2h, Wg2, bg2_)
    x3h, _ = _conv_kernel(h2, nidx3, eidx3, recip_e, recip_v, t3h)
    x3 = jnp.concatenate([x3h[0, :NV], x3h[1, :NV]], axis=1)
    return (x3, topo3)


# speedup vs baseline: 1.0545x; 1.0545x over previous
import dataclasses
import functools

import jax
import jax.numpy as jnp
from jax import lax
from jax.experimental import pallas as pl
from jax.experimental.pallas import tpu as pltpu
from jax.experimental.pallas import tpu_sc as plsc

NV = 10000
NE = 10000
NNZ = 320000
D = 128
DH = 64
NS = 16
CHUNK = 80
PAIRS_PER_TILE = NNZ // NS
CHUNKS_PER_TILE = PAIRS_PER_TILE // CHUNK
NPAD = 10240
STRIPE = NPAD // NS
NROWS = NPAD // 16
RB = 1000

_MESH = plsc.VectorSubcoreMesh(core_axis_name="c", subcore_axis_name="s")

_SC_PARAMS = pltpu.CompilerParams()
for _f, _v in (("needs_layout_passes", False), ("use_tc_tiling_on_sc", False)):
    if _f in pltpu.CompilerParams.__dataclass_fields__:
        _SC_PARAMS = dataclasses.replace(_SC_PARAMS, **{_f: _v})


@functools.partial(
    pl.kernel,
    out_type=jax.ShapeDtypeStruct((2, NROWS, 16), jnp.float32),
    mesh=_MESH,
    compiler_params=_SC_PARAMS,
    scratch_types=[
        pltpu.VMEM((CHUNKS_PER_TILE, CHUNK), jnp.int32),
        pltpu.VMEM((NROWS, 16), jnp.float32),
        pltpu.VMEM((NROWS, 16), jnp.float32),
        pltpu.VMEM((5, 128), jnp.int32),
        pltpu.VMEM_SHARED((NROWS, 16), jnp.float32),
        pltpu.VMEM_SHARED((NROWS, 16), jnp.float32),
    ],
)
def _deg_kernel(nidx3, eidx3, rout, idx_t, dega_t, degb_t, ramp_t,
                acca_s, accb_s):
    c = lax.axis_index("c")
    s = lax.axis_index("s")
    z16 = jnp.zeros((16,), jnp.float32)
    ones16 = jnp.ones((16,), jnp.float32)
    iota16 = lax.iota(jnp.int32, 16)

    @pl.loop(0, NROWS)
    def _(r):
        dega_t[r, :] = z16
        degb_t[r, :] = z16

    pltpu.sync_copy(dega_t.at[pl.ds(0, 40)], acca_s.at[pl.ds(s * 40, 40)])
    pltpu.sync_copy(degb_t.at[pl.ds(0, 40)], accb_s.at[pl.ds(s * 40, 40)])
    plsc.subcore_barrier()

    def _hist(idx3, deg_t):
        pltpu.sync_copy(idx3.at[s], idx_t)

        @pl.loop(0, CHUNKS_PER_TILE)
        def _(r):
            for k in range(CHUNK // 16):
                v = idx_t[r, pl.ds(k * 16, 16)]
                plsc.addupdate_scatter(
                    deg_t,
                    [lax.shift_right_logical(v, 4), lax.bitwise_and(v, 15)],
                    ones16,
                )

    _hist(eidx3, dega_t)
    _hist(nidx3, degb_t)

    for rr in range(5):
        for jj in range(8):
            ramp_t[rr, pl.ds(jj * 16, 16)] = iota16 + (rr * 128 + jj * 16)
    for rr in range(5):
        pltpu.sync_copy(dega_t.at[pl.ds(rr * 128, 128)],
                        acca_s.at[ramp_t.at[rr]], add=True)
        pltpu.sync_copy(degb_t.at[pl.ds(rr * 128, 128)],
                        accb_s.at[ramp_t.at[rr]], add=True)
    plsc.subcore_barrier()

    def _recip(acc_s, deg_t, kind):
        pltpu.sync_copy(acc_s.at[pl.ds(s * 40, 40)], deg_t.at[pl.ds(0, 40)])

        @pl.loop(0, 40)
        def _(r):
            deg_t[r, :] = 1.0 / jnp.maximum(deg_t[r, :], 1.0)

        pltpu.sync_copy(deg_t.at[pl.ds(0, 40)],
                        rout.at[kind].at[pl.ds(s * 40, 40)])

    @pl.when(c == 0)
    def _():
        _recip(acca_s, dega_t, 0)

    @pl.when(c == 1)
    def _():
        _recip(accb_s, degb_t, 1)


@functools.partial(
    pl.kernel,
    out_type=[jax.ShapeDtypeStruct((2, NPAD, DH), jnp.float32),
              jax.ShapeDtypeStruct((2, NPAD, DH), jnp.float32)],
    mesh=_MESH,
    compiler_params=_SC_PARAMS,
    scratch_types=[
        pltpu.VMEM((CHUNKS_PER_TILE, CHUNK), jnp.int32),
        pltpu.VMEM((CHUNKS_PER_TILE, CHUNK), jnp.int32),
        pltpu.VMEM((CHUNK, DH), jnp.float32),
        pltpu.VMEM((CHUNK, DH), jnp.float32),
        pltpu.VMEM((CHUNK, DH), jnp.float32),
        pltpu.VMEM((CHUNK, DH), jnp.float32),
        pltpu.VMEM((CHUNK, DH), jnp.float32),
        pltpu.VMEM((CHUNK, DH), jnp.float32),
        pltpu.VMEM((CHUNK, DH), jnp.float32),
        pltpu.VMEM((CHUNK, DH), jnp.float32),
        pltpu.VMEM((1, STRIPE), jnp.float32),
        pltpu.VMEM_SHARED((NPAD, DH), jnp.float32),
    ] + [pltpu.SemaphoreType.DMA] * 11,
)
def _conv_kernel(h2, nidx3, eidx3, recip_e, recip_v, out2, e_out,
                 nidx_t, eidx_t, r0, r1, r2, r3, r4, tmp_t, ttmp_t, zeros_t,
                 recip_t, acc, g0, g1, g2, g3, g4, s0, s1, s2, s3, s4, zs):
    c = lax.axis_index("c")
    s = lax.axis_index("s")
    z16 = jnp.zeros((16,), jnp.float32)
    z16i = jnp.zeros((16,), jnp.int32)
    NCH = STRIPE // CHUNK

    @pl.loop(0, CHUNK)
    def _(r):
        for k in range(DH // 16):
            zeros_t[r, pl.ds(k * 16, 16)] = z16

    def _zero_chunk_start(j):
        pltpu.async_copy(zeros_t, acc.at[pl.ds(s * STRIPE + j * CHUNK, CHUNK)],
                         zs)

    def _zero_drain(n):
        for _ in range(n):
            pltpu.make_async_copy(
                zeros_t, acc.at[pl.ds(s * STRIPE, CHUNK)], zs).wait()

    for j in range(NCH):
        _zero_chunk_start(j)
    pltpu.sync_copy(nidx3.at[s], nidx_t)
    pltpu.sync_copy(eidx3.at[s], eidx_t)
    _zero_drain(NCH)
    plsc.subcore_barrier()

    NB = 5
    BUFS = (r0, r1, r2, r3, r4)
    GSEMS = (g0, g1, g2, g3, g4)
    SSEMS = (s0, s1, s2, s3, s4)
    NITER = CHUNKS_PER_TILE // NB

    def _pipelined_pass(src_ref, gidx_t, dst_acc, sidx_t):
        def g_start(b, k):
            pltpu.async_copy(src_ref.at[gidx_t.at[k]], BUFS[b], GSEMS[b])

        def g_wait(b):
            pltpu.make_async_copy(
                src_ref.at[gidx_t.at[0]], BUFS[b], GSEMS[b]).wait()

        def s_start(b, k):
            pltpu.async_copy(BUFS[b], dst_acc.at[sidx_t.at[k]], SSEMS[b],
                             add=True)

        def s_wait(b):
            pltpu.make_async_copy(
                BUFS[b], dst_acc.at[sidx_t.at[0]], SSEMS[b]).wait()

        for b in range(NB):
            g_start(b, b)

        @pl.loop(0, NITER)
        def _(i):
            k = i * NB
            for b in range(NB):
                g_wait(b)
                s_start(b, k + b)

            @pl.when(i < NITER - 1)
            def _():
                for b in range(NB):
                    s_wait(b)
                    g_start(b, k + NB + b)

        for b in range(NB):
            s_wait(b)

    def _scale_rows(buf, j):
        @pl.loop(0, CHUNK)
        def _(r):
            rec = plsc.load_gather(
                recip_t, [z16i, jnp.full((16,), j * CHUNK + r, jnp.int32)])
            for k in range(DH // 16):
                buf[r, pl.ds(k * 16, 16)] = buf[r, pl.ds(k * 16, 16)] * rec

    def _combine_rows(buf, tbuf, j):
        @pl.loop(0, CHUNK)
        def _(r):
            rec = plsc.load_gather(
                recip_t, [z16i, jnp.full((16,), j * CHUNK + r, jnp.int32)])
            for k in range(DH // 16):
                v = buf[r, pl.ds(k * 16, 16)] * rec
                t = tbuf[r, pl.ds(k * 16, 16)]
                buf[r, pl.ds(k * 16, 16)] = jnp.maximum(v + v * t, 0.0)

    def _normalize_phase(dst_ref, fuse_zero, topo_src=None):
        bufs = (tmp_t, ttmp_t)
        tbufs = (r0, r1)
        rsems = (g0, g1)
        tsems = (g2, g3)
        wsems = (s0, s1)

        def chunk_slice(ref, j):
            return ref.at[pl.ds(s * STRIPE + j * CHUNK, CHUNK)]

        def r_start(j):
            pltpu.async_copy(chunk_slice(acc, j), bufs[j % 2], rsems[j % 2])
            if topo_src is not None:
                pltpu.async_copy(chunk_slice(topo_src, j), tbufs[j % 2],
                                 tsems[j % 2])

        def r_wait(j):
            pltpu.make_async_copy(
                chunk_slice(acc, j), bufs[j % 2], rsems[j % 2]).wait()
            if topo_src is not None:
                pltpu.make_async_copy(
                    chunk_slice(topo_src, j), tbufs[j % 2],
                    tsems[j % 2]).wait()

        def w_start(j):
            pltpu.async_copy(bufs[j % 2], chunk_slice(dst_ref, j),
                             wsems[j % 2])

        def w_wait(j):
            pltpu.make_async_copy(bufs[j % 2], chunk_slice(dst_ref, j),
                                  wsems[j % 2]).wait()

        r_start(0)
        for j in range(NCH):
            r_wait(j)
            if fuse_zero:
                _zero_chunk_start(j)
            if j + 1 < NCH:
                if j >= 1:
                    w_wait(j - 1)
                r_start(j + 1)
            if topo_src is None:
                _scale_rows(bufs[j % 2], j)
            else:
                _combine_rows(bufs[j % 2], tbufs[j % 2], j)
            w_start(j)
        w_wait(NCH - 2)
        w_wait(NCH - 1)
        if fuse_zero:
            _zero_drain(NCH)

    hc = h2.at[c]
    _pipelined_pass(hc, nidx_t, acc, eidx_t)
    plsc.subcore_barrier()

    ec = e_out.at[c]
    pltpu.sync_copy(recip_e.at[s], recip_t)
    _normalize_phase(ec, fuse_zero=True)
    plsc.subcore_barrier()

    _pipelined_pass(ec, eidx_t, acc, nidx_t)
    plsc.subcore_barrier()

    pltpu.sync_copy(recip_v.at[s], recip_t)
    _normalize_phase(out2.at[c], fuse_zero=False)


def _mm(a, b):
    return jnp.dot(a, b, preferred_element_type=jnp.float32,
                   precision=lax.Precision.HIGHEST)


def _t0_body(pd_ref, tW_ref, tb_ref, x_ref, Wg_ref, bg_ref, topo_ref, h2_ref):
    topo = jnp.maximum(_mm(pd_ref[...], tW_ref[...]) + tb_ref[...], 0.0)
    topo_ref[...] = topo
    h = _mm(x_ref[...], Wg_ref[...]) + bg_ref[...]
    h2_ref[0] = h[:, :DH]
    h2_ref[1] = h[:, DH:]


def _tmid_body(o2_ref, topo_ref, Wt_ref, bt_ref, Wg_ref, bg_ref,
               topo_out_ref, h2_ref):
    topo = _mm(topo_ref[...], Wt_ref[...]) + bt_ref[...]
    topo_out_ref[...] = topo
    conv = jnp.concatenate([o2_ref[0], o2_ref[1]], axis=1)
    xl = jnp.maximum(conv * (1.0 + topo), 0.0)
    h = _mm(xl, Wg_ref[...]) + bg_ref[...]
    h2_ref[0] = h[:, :DH]
    h2_ref[1] = h[:, DH:]


def _tfin_body(o2_ref, topo_ref, Wt_ref, bt_ref, x_ref, topo_out_ref):
    topo = _mm(topo_ref[...], Wt_ref[...]) + bt_ref[...]
    conv = jnp.concatenate([o2_ref[0], o2_ref[1]], axis=1)
    x_ref[...] = jnp.maximum(conv * (1.0 + topo), 0.0)
    topo_out_ref[...] = topo


_b_mat = pl.BlockSpec((D, D), lambda i: (0, 0))
_b_bias = pl.BlockSpec((1, D), lambda i: (0, 0))
_b_row = pl.BlockSpec((RB, D), lambda i: (i, 0))
_b_h2 = pl.BlockSpec((2, RB, DH), lambda i: (0, i, 0))

_PADH = jax.ShapeDtypeStruct((2, NPAD, DH), jnp.float32)

_t0 = pl.pallas_call(
    _t0_body,
    grid=(NV // RB,),
    in_specs=[pl.BlockSpec((RB, 5), lambda i: (i, 0)),
              pl.BlockSpec((5, D), lambda i: (0, 0)),
              _b_bias, _b_row, _b_mat, _b_bias],
    out_specs=[_b_row, _b_h2],
    out_shape=[jax.ShapeDtypeStruct((NV, D), jnp.float32), _PADH],
)

_tmid = pl.pallas_call(
    _tmid_body,
    grid=(NV // RB,),
    in_specs=[_b_h2, _b_row, _b_mat, _b_bias, _b_mat, _b_bias],
    out_specs=[_b_row, _b_h2],
    out_shape=[jax.ShapeDtypeStruct((NV, D), jnp.float32), _PADH],
)

_tfin = pl.pallas_call(
    _tfin_body,
    grid=(NV // RB,),
    in_specs=[_b_h2, _b_row, _b_mat, _b_bias],
    out_specs=[_b_row, _b_row],
    out_shape=[jax.ShapeDtypeStruct((NV, D), jnp.float32),
               jax.ShapeDtypeStruct((NV, D), jnp.float32)],
)


def kernel(x, hg, pd, tW, tb, Wg0, bg0, Wt0, bt0, Wg1, bg1, Wt1, bt1,
           Wg2, bg2, Wt2, bt2):
    nidx3 = hg[0].reshape(NS, CHUNKS_PER_TILE, CHUNK)
    eidx3 = hg[1].reshape(NS, CHUNKS_PER_TILE, CHUNK)
    rout = _deg_kernel(nidx3, eidx3)
    recip_e = rout[0].reshape(NS, 1, STRIPE)
    recip_v = rout[1].reshape(NS, 1, STRIPE)

    tb_ = tb.reshape(1, D)
    bg0_, bt0_ = bg0.reshape(1, D), bt0.reshape(1, D)
    bg1_, bt1_ = bg1.reshape(1, D), bt1.reshape(1, D)
    bg2_, bt2_ = bg2.reshape(1, D), bt2.reshape(1, D)

    topo0, h2 = _t0(pd, tW, tb_, x, Wg0, bg0_)
    out2, _ = _conv_kernel(h2, nidx3, eidx3, recip_e, recip_v)
    topo1, h2 = _tmid(out2, topo0, Wt0, bt0_, Wg1, bg1_)
    out2, _ = _conv_kernel(h2, nidx3, eidx3, recip_e, recip_v)
    topo2, h2 = _tmid(out2, topo1, Wt1, bt1_, Wg2, bg2_)
    out2, _ = _conv_kernel(h2, nidx3, eidx3, recip_e, recip_v)
    x3, topo3 = _tfin(out2, topo2, Wt2, bt2_)
    return (x3, topo3)


# speedup vs baseline: 1.1025x; 1.0456x over previous
import dataclasses
import functools

import jax
import jax.numpy as jnp
from jax import lax
from jax.experimental import pallas as pl
from jax.experimental.pallas import tpu as pltpu
from jax.experimental.pallas import tpu_sc as plsc

NV = 10000
NE = 10000
NNZ = 320000
D = 128
DH = 64
NS = 16
CHUNK = 80
PAIRS_PER_TILE = NNZ // NS
CHUNKS_PER_TILE = PAIRS_PER_TILE // CHUNK
NPAD = 10240
STRIPE = NPAD // NS
NROWS = NPAD // 16
RB = 1000

_MESH = plsc.VectorSubcoreMesh(core_axis_name="c", subcore_axis_name="s")

_SC_PARAMS = pltpu.CompilerParams()
for _f, _v in (("needs_layout_passes", False), ("use_tc_tiling_on_sc", False)):
    if _f in pltpu.CompilerParams.__dataclass_fields__:
        _SC_PARAMS = dataclasses.replace(_SC_PARAMS, **{_f: _v})


@functools.partial(
    pl.kernel,
    out_type=jax.ShapeDtypeStruct((2, NROWS, 16), jnp.float32),
    mesh=_MESH,
    compiler_params=_SC_PARAMS,
    scratch_types=[
        pltpu.VMEM((CHUNKS_PER_TILE, CHUNK), jnp.int32),
        pltpu.VMEM((NROWS, 16), jnp.float32),
        pltpu.VMEM((NROWS, 16), jnp.float32),
        pltpu.VMEM((5, 128), jnp.int32),
        pltpu.VMEM_SHARED((NROWS, 16), jnp.float32),
        pltpu.VMEM_SHARED((NROWS, 16), jnp.float32),
    ],
)
def _deg_kernel(nidx3, eidx3, rout, idx_t, dega_t, degb_t, ramp_t,
                acca_s, accb_s):
    c = lax.axis_index("c")
    s = lax.axis_index("s")
    z16 = jnp.zeros((16,), jnp.float32)
    ones16 = jnp.ones((16,), jnp.float32)
    iota16 = lax.iota(jnp.int32, 16)

    @pl.loop(0, NROWS)
    def _(r):
        dega_t[r, :] = z16
        degb_t[r, :] = z16

    pltpu.sync_copy(dega_t.at[pl.ds(0, 40)], acca_s.at[pl.ds(s * 40, 40)])
    pltpu.sync_copy(degb_t.at[pl.ds(0, 40)], accb_s.at[pl.ds(s * 40, 40)])
    plsc.subcore_barrier()

    def _hist(idx3, deg_t):
        pltpu.sync_copy(idx3.at[s], idx_t)

        @pl.loop(0, CHUNKS_PER_TILE)
        def _(r):
            for k in range(CHUNK // 16):
                v = idx_t[r, pl.ds(k * 16, 16)]
                plsc.addupdate_scatter(
                    deg_t,
                    [lax.shift_right_logical(v, 4), lax.bitwise_and(v, 15)],
                    ones16,
                )

    _hist(eidx3, dega_t)
    _hist(nidx3, degb_t)

    for rr in range(5):
        for jj in range(8):
            ramp_t[rr, pl.ds(jj * 16, 16)] = iota16 + (rr * 128 + jj * 16)
    for rr in range(5):
        pltpu.sync_copy(dega_t.at[pl.ds(rr * 128, 128)],
                        acca_s.at[ramp_t.at[rr]], add=True)
        pltpu.sync_copy(degb_t.at[pl.ds(rr * 128, 128)],
                        accb_s.at[ramp_t.at[rr]], add=True)
    plsc.subcore_barrier()

    def _recip(acc_s, deg_t, kind):
        pltpu.sync_copy(acc_s.at[pl.ds(s * 40, 40)], deg_t.at[pl.ds(0, 40)])

        @pl.loop(0, 40)
        def _(r):
            deg_t[r, :] = 1.0 / jnp.maximum(deg_t[r, :], 1.0)

        pltpu.sync_copy(deg_t.at[pl.ds(0, 40)],
                        rout.at[kind].at[pl.ds(s * 40, 40)])

    @pl.when(c == 0)
    def _():
        _recip(acca_s, dega_t, 0)

    @pl.when(c == 1)
    def _():
        _recip(accb_s, degb_t, 1)


@functools.partial(
    pl.kernel,
    out_type=[jax.ShapeDtypeStruct((NPAD, D), jnp.float32),
              jax.ShapeDtypeStruct((2, NPAD, DH), jnp.float32)],
    mesh=_MESH,
    compiler_params=_SC_PARAMS,
    scratch_types=[
        pltpu.VMEM((CHUNKS_PER_TILE, CHUNK), jnp.int32),
        pltpu.VMEM((CHUNKS_PER_TILE, CHUNK), jnp.int32),
    ] + [pltpu.VMEM((CHUNK, DH), jnp.float32)] * 5
      + [
        pltpu.VMEM((CHUNK, DH), jnp.float32),
        pltpu.VMEM((CHUNK, DH), jnp.float32),
        pltpu.VMEM((CHUNK, DH), jnp.float32),
        pltpu.VMEM((1, STRIPE), jnp.float32),
        pltpu.VMEM_SHARED((NPAD, DH), jnp.float32),
    ] + [pltpu.SemaphoreType.DMA] * 11,
)
def _conv_kernel(h2, nidx3, eidx3, recip_e, recip_v, out2, e_out,
                 nidx_t, eidx_t, *scr):
    RING = scr[0:5]
    tmp_t, ttmp_t, zeros_t, recip_t, acc = scr[5:10]
    RSEMS = scr[10:15]
    WSEMS = scr[15:20]
    zs = scr[20]
    g0, g1, g2, g3 = RSEMS[0], RSEMS[1], RSEMS[2], RSEMS[3]
    s0, s1 = WSEMS[0], WSEMS[1]
    c = lax.axis_index("c")
    s = lax.axis_index("s")
    z16 = jnp.zeros((16,), jnp.float32)
    z16i = jnp.zeros((16,), jnp.int32)
    NCH = STRIPE // CHUNK

    @pl.loop(0, CHUNK)
    def _(r):
        for k in range(DH // 16):
            zeros_t[r, pl.ds(k * 16, 16)] = z16

    def _zero_chunk_start(j):
        pltpu.async_copy(zeros_t, acc.at[pl.ds(s * STRIPE + j * CHUNK, CHUNK)],
                         zs)

    def _zero_drain(n):
        for _ in range(n):
            pltpu.make_async_copy(
                zeros_t, acc.at[pl.ds(s * STRIPE, CHUNK)], zs).wait()

    for j in range(NCH):
        _zero_chunk_start(j)
    pltpu.sync_copy(nidx3.at[s], nidx_t)
    pltpu.sync_copy(eidx3.at[s], eidx_t)
    _zero_drain(NCH)
    plsc.subcore_barrier()

    NB = 5
    BUFS = RING
    GSEMS = RSEMS
    SSEMS = WSEMS
    NITER = CHUNKS_PER_TILE // NB

    def _pipelined_pass(src_ref, gidx_t, dst_acc, sidx_t):
        def g_start(b, k):
            pltpu.async_copy(src_ref.at[gidx_t.at[k]], BUFS[b], GSEMS[b])

        def g_wait(b):
            pltpu.make_async_copy(
                src_ref.at[gidx_t.at[0]], BUFS[b], GSEMS[b]).wait()

        def s_start(b, k):
            pltpu.async_copy(BUFS[b], dst_acc.at[sidx_t.at[k]], SSEMS[b],
                             add=True)

        def s_wait(b):
            pltpu.make_async_copy(
                BUFS[b], dst_acc.at[sidx_t.at[0]], SSEMS[b]).wait()

        for b in range(NB):
            g_start(b, b)

        @pl.loop(0, NITER)
        def _(i):
            k = i * NB
            for b in range(NB):
                g_wait(b)
                s_start(b, k + b)

            @pl.when(i < NITER - 1)
            def _():
                for b in range(NB):
                    s_wait(b)
                    g_start(b, k + NB + b)

        for b in range(NB):
            s_wait(b)

    def _scale_rows(buf, j):
        @pl.loop(0, CHUNK)
        def _(r):
            rec = plsc.load_gather(
                recip_t, [z16i, jnp.full((16,), j * CHUNK + r, jnp.int32)])
            for k in range(DH // 16):
                buf[r, pl.ds(k * 16, 16)] = buf[r, pl.ds(k * 16, 16)] * rec

    def _combine_rows(buf, tbuf, j):
        @pl.loop(0, CHUNK)
        def _(r):
            rec = plsc.load_gather(
                recip_t, [z16i, jnp.full((16,), j * CHUNK + r, jnp.int32)])
            for k in range(DH // 16):
                v = buf[r, pl.ds(k * 16, 16)] * rec
                t = tbuf[r, pl.ds(k * 16, 16)]
                buf[r, pl.ds(k * 16, 16)] = jnp.maximum(v + v * t, 0.0)

    def _normalize_phase(dst_ref, fuse_zero, topo_src=None):
        bufs = (tmp_t, ttmp_t)
        tbufs = (RING[0], RING[1])
        rsems = (g0, g1)
        tsems = (g2, g3)
        wsems = (s0, s1)

        def chunk_slice(ref, j):
            return ref.at[pl.ds(s * STRIPE + j * CHUNK, CHUNK)]

        def r_start(j):
            pltpu.async_copy(chunk_slice(acc, j), bufs[j % 2], rsems[j % 2])
            if topo_src is not None:
                pltpu.async_copy(chunk_slice(topo_src, j), tbufs[j % 2],
                                 tsems[j % 2])

        def r_wait(j):
            pltpu.make_async_copy(
                chunk_slice(acc, j), bufs[j % 2], rsems[j % 2]).wait()
            if topo_src is not None:
                pltpu.make_async_copy(
                    chunk_slice(topo_src, j), tbufs[j % 2],
                    tsems[j % 2]).wait()

        def w_start(j):
            pltpu.async_copy(bufs[j % 2], chunk_slice(dst_ref, j),
                             wsems[j % 2])

        def w_wait(j):
            pltpu.make_async_copy(bufs[j % 2], chunk_slice(dst_ref, j),
                                  wsems[j % 2]).wait()

        r_start(0)
        for j in range(NCH):
            r_wait(j)
            if fuse_zero:
                _zero_chunk_start(j)
            if j + 1 < NCH:
                if j >= 1:
                    w_wait(j - 1)
                r_start(j + 1)
            if topo_src is None:
                _scale_rows(bufs[j % 2], j)
            else:
                _combine_rows(bufs[j % 2], tbufs[j % 2], j)
            w_start(j)
        w_wait(NCH - 2)
        w_wait(NCH - 1)
        if fuse_zero:
            _zero_drain(NCH)

    hc = h2.at[c]
    _pipelined_pass(hc, nidx_t, acc, eidx_t)
    plsc.subcore_barrier()

    ec = e_out.at[c]
    pltpu.sync_copy(recip_e.at[s], recip_t)
    _normalize_phase(ec, fuse_zero=True)
    plsc.subcore_barrier()

    _pipelined_pass(ec, eidx_t, acc, nidx_t)
    plsc.subcore_barrier()

    pltpu.sync_copy(recip_v.at[s], recip_t)
    _normalize_phase(out2.at[:, pl.ds(c * DH, DH)], fuse_zero=False)


def _mm(a, b):
    return jnp.dot(a, b, preferred_element_type=jnp.float32,
                   precision=lax.Precision.HIGHEST)


def _t0_body(pd_ref, tW_ref, tb_ref, x_ref, Wg_ref, bg_ref, topo_ref, h2_ref):
    topo = jnp.maximum(_mm(pd_ref[...], tW_ref[...]) + tb_ref[...], 0.0)
    topo_ref[...] = topo
    h = _mm(x_ref[...], Wg_ref[...]) + bg_ref[...]
    h2_ref[0] = h[:, :DH]
    h2_ref[1] = h[:, DH:]


def _tmid_body(o_ref, topo_ref, Wt_ref, bt_ref, Wg_ref, bg_ref,
               topo_out_ref, h2_ref):
    topo = _mm(topo_ref[...], Wt_ref[...]) + bt_ref[...]
    topo_out_ref[...] = topo
    xl = jnp.maximum(o_ref[...] * (1.0 + topo), 0.0)
    h = _mm(xl, Wg_ref[...]) + bg_ref[...]
    h2_ref[0] = h[:, :DH]
    h2_ref[1] = h[:, DH:]


def _tfin_body(o_ref, topo_ref, Wt_ref, bt_ref, x_ref, topo_out_ref):
    topo = _mm(topo_ref[...], Wt_ref[...]) + bt_ref[...]
    x_ref[...] = jnp.maximum(o_ref[...] * (1.0 + topo), 0.0)
    topo_out_ref[...] = topo


_b_mat = pl.BlockSpec((D, D), lambda i: (0, 0))
_b_bias = pl.BlockSpec((1, D), lambda i: (0, 0))
_b_row = pl.BlockSpec((RB, D), lambda i: (i, 0))
_b_h2 = pl.BlockSpec((2, RB, DH), lambda i: (0, i, 0))

_PADH = jax.ShapeDtypeStruct((2, NPAD, DH), jnp.float32)

_t0 = pl.pallas_call(
    _t0_body,
    grid=(NV // RB,),
    in_specs=[pl.BlockSpec((RB, 5), lambda i: (i, 0)),
              pl.BlockSpec((5, D), lambda i: (0, 0)),
              _b_bias, _b_row, _b_mat, _b_bias],
    out_specs=[_b_row, _b_h2],
    out_shape=[jax.ShapeDtypeStruct((NV, D), jnp.float32), _PADH],
)

_tmid = pl.pallas_call(
    _tmid_body,
    grid=(NV // RB,),
    in_specs=[_b_row, _b_row, _b_mat, _b_bias, _b_mat, _b_bias],
    out_specs=[_b_row, _b_h2],
    out_shape=[jax.ShapeDtypeStruct((NV, D), jnp.float32), _PADH],
)

_tfin = pl.pallas_call(
    _tfin_body,
    grid=(NV // RB,),
    in_specs=[_b_row, _b_row, _b_mat, _b_bias],
    out_specs=[_b_row, _b_row],
    out_shape=[jax.ShapeDtypeStruct((NV, D), jnp.float32),
               jax.ShapeDtypeStruct((NV, D), jnp.float32)],
)


def kernel(x, hg, pd, tW, tb, Wg0, bg0, Wt0, bt0, Wg1, bg1, Wt1, bt1,
           Wg2, bg2, Wt2, bt2):
    nidx3 = hg[0].reshape(NS, CHUNKS_PER_TILE, CHUNK)
    eidx3 = hg[1].reshape(NS, CHUNKS_PER_TILE, CHUNK)
    rout = _deg_kernel(nidx3, eidx3)
    recip_e = rout[0].reshape(NS, 1, STRIPE)
    recip_v = rout[1].reshape(NS, 1, STRIPE)

    tb_ = tb.reshape(1, D)
    bg0_, bt0_ = bg0.reshape(1, D), bt0.reshape(1, D)
    bg1_, bt1_ = bg1.reshape(1, D), bt1.reshape(1, D)
    bg2_, bt2_ = bg2.reshape(1, D), bt2.reshape(1, D)

    topo0, h2 = _t0(pd, tW, tb_, x, Wg0, bg0_)
    out2, _ = _conv_kernel(h2, nidx3, eidx3, recip_e, recip_v)
    topo1, h2 = _tmid(out2, topo0, Wt0, bt0_, Wg1, bg1_)
    out2, _ = _conv_kernel(h2, nidx3, eidx3, recip_e, recip_v)
    topo2, h2 = _tmid(out2, topo1, Wt1, bt1_, Wg2, bg2_)
    out2, _ = _conv_kernel(h2, nidx3, eidx3, recip_e, recip_v)
    x3, topo3 = _tfin(out2, topo2, Wt2, bt2_)
    return (x3, topo3)
